# baseline (speedup 1.0000x reference)
import jax
import jax.numpy as jnp
from jax import lax
from jax.experimental import pallas as pl
from jax.experimental.pallas import tpu as pltpu
from jax.experimental.pallas import tpu_sc as plsc

N = 131072
DIM = 256
NUM_EDGES = 101
EDGE_PAD = 128
NC, NS, L = 2, 16, 16
NW = NC * NS
BPW = N // NW
CH = 128
NCH = BPW // CH
SUP = 1024
NSUP = BPW // SUP


def _sc_body(time_hbm, edges_hbm, table_hbm, out_hbm,
             tv, ev, tab, buf0, buf1, sidx, sem0, sem1):
    wid = lax.axis_index("s") * NC + lax.axis_index("c")
    base = wid * BPW
    pltpu.sync_copy(time_hbm.at[pl.ds(base, BPW)], tv)
    pltpu.sync_copy(edges_hbm, ev)
    pltpu.sync_copy(table_hbm, tab)

    def search16(off):
        t = tv[pl.ds(off, L)]
        pos = jnp.zeros((L,), jnp.int32)
        for s in (64, 32, 16, 8, 4, 2, 1):
            cand = pos + s
            e = plsc.load_gather(ev, [cand - 1])
            pos = jnp.where(e < t, cand, pos)
        return jnp.minimum(pos, NUM_EDGES - 1) * DIM

    def search_super(s):
        def step(q, carry):
            pv = search16(s * SUP + q * L)
            for l in range(L):
                sidx[q * L + l] = pv[l]
            return carry
        lax.fori_loop(0, SUP // L, step, 0)

    def build(j, buf):
        def row_step(r, carry):
            i = sidx[j * CH + r]
            for g in range(DIM // L):
                buf[pl.ds(r * DIM + g * L, L)] = tab[pl.ds(i + g * L, L)]
            return carry
        lax.fori_loop(0, CH, row_step, 0)

    def fire(c, buf, sem):
        return pltpu.async_copy(
            buf, out_hbm.at[pl.ds((base + c * CH) * DIM, CH * DIM)], sem)

    bufs = (buf0, buf1)
    sems = (sem0, sem1)
    pending = [None, None]
    for s in range(NSUP):
        search_super(s)
        for j in range(NCH // NSUP):
            c = s * (NCH // NSUP) + j
            bb = c % 2
            if pending[bb] is not None:
                pending[bb].wait()
            build(j, bufs[bb])
            pending[bb] = fire(c, bufs[bb], sems[bb])
    pending[0].wait()
    pending[1].wait()


def _build():
    mesh = plsc.VectorSubcoreMesh(core_axis_name="c", subcore_axis_name="s")
    return pl.kernel(
        _sc_body,
        out_type=jax.ShapeDtypeStruct((N * DIM,), jnp.float32),
        mesh=mesh,
        compiler_params=pltpu.CompilerParams(needs_layout_passes=False),
        scratch_types=[
            pltpu.VMEM((BPW,), jnp.float32),
            pltpu.VMEM((EDGE_PAD,), jnp.float32),
            pltpu.VMEM((NUM_EDGES * DIM,), jnp.float32),
            pltpu.VMEM((CH * DIM,), jnp.float32),
            pltpu.VMEM((CH * DIM,), jnp.float32),
            pltpu.SMEM((SUP,), jnp.int32),
            pltpu.SemaphoreType.DMA,
            pltpu.SemaphoreType.DMA,
        ],
    )


def _impl(time_value, bin_edges, embed_table):
    pad = jnp.full((EDGE_PAD - NUM_EDGES,), jnp.inf, dtype=jnp.float32)
    edges_pad = jnp.concatenate([bin_edges.astype(jnp.float32), pad])
    flat = _build()(time_value, edges_pad, embed_table.reshape(-1))
    return flat.reshape(N, DIM)


_jitted = jax.jit(_impl)


def kernel(time_value, bin_edges, embed_table):
    return _jitted(time_value, bin_edges, embed_table)
